# baseline (speedup 1.0000x reference)
import dataclasses
import functools

import jax
import jax.numpy as jnp
import numpy as np
from jax import lax
from jax.experimental import pallas as pl
from jax.experimental.pallas import tpu as pltpu
from jax.experimental.pallas import tpu_sc as plsc

CUTOFF = 8.0
N_NODES = 10000
N_EDGES = 160000
N_TRIPLETS = 320000
HIDDEN = 128
NUM_RADIAL = 16
NUM_SPHERICAL = 6

_MESH = dict(core_axis_name="c", subcore_axis_name="s")


def _sc_gather(table, idx, D, N, W=128):
    idx2 = idx.reshape(1, N)

    @functools.partial(
        pl.kernel,
        out_type=jax.ShapeDtypeStruct((N, D), table.dtype),
        mesh=plsc.VectorSubcoreMesh(**_MESH),
    )
    def k(tab_hbm, i_hbm, o_hbm):
        def body(i_vmem, o_vmem):
            pltpu.sync_copy(tab_hbm.at[i_vmem.at[0]], o_vmem)

        pltpu.emit_pipeline(
            body,
            grid=(N // W,),
            in_specs=[pl.BlockSpec((1, W), lambda g: (0, g))],
            out_specs=[pl.BlockSpec((W, D), lambda g: (g, 0))],
            core_axis_name=("c", "s"),
            dimension_semantics=(pltpu.PARALLEL,),
        )(i_hbm, o_hbm)

    return k(table, idx2)


_D_HALF = 80000
_D_PER = 10000


def _sc_dist_gather(dist, idx):
    cp = pltpu.CompilerParams()
    if "needs_layout_passes" in pltpu.CompilerParams.__dataclass_fields__:
        cp = dataclasses.replace(cp, needs_layout_passes=False)

    @functools.partial(
        pl.kernel,
        out_type=jax.ShapeDtypeStruct((N_TRIPLETS,), jnp.float32),
        mesh=plsc.VectorSubcoreMesh(**_MESH),
        compiler_params=cp,
        scratch_types=[
            pltpu.VMEM((_D_HALF,), jnp.float32),
            pltpu.VMEM((_D_PER,), jnp.int32),
            pltpu.VMEM((_D_PER,), jnp.float32),
        ],
    )
    def k(d_hbm, i_hbm, o_hbm, tab, idxb, outb):
        w = lax.axis_index("c") * 16 + lax.axis_index("s")
        base = w * _D_PER
        pltpu.sync_copy(i_hbm.at[pl.ds(base, _D_PER)], idxb)
        for h in range(2):
            pltpu.sync_copy(d_hbm.at[pl.ds(h * _D_HALF, _D_HALF)], tab)

            @pl.loop(0, _D_PER, step=16)
            def _(o):
                v = idxb[pl.ds(o, 16)] - h * _D_HALF
                m = (v >= 0) & (v < _D_HALF)
                vc = jnp.clip(v, 0, _D_HALF - 1)
                g = plsc.load_gather(tab, [vc])
                r = jnp.where(m, g, 0.0)
                if h == 0:
                    outb[pl.ds(o, 16)] = r
                else:
                    outb[pl.ds(o, 16)] = outb[pl.ds(o, 16)] + r

        pltpu.sync_copy(outb, o_hbm.at[pl.ds(base, _D_PER)])

    return k(dist, idx)


_S_ROWS = 80000
_S_STRIPE = 5008
_S_ACC = 80128
_S_CHUNK = 1280
_S_PER = 20000
_S_TAIL = _S_PER - 15 * _S_CHUNK
_S_COLS = 32
_S_ZROWS = 313


def _sc_scatter_add(u, idx):
    @functools.partial(
        pl.kernel,
        out_type=jax.ShapeDtypeStruct((N_EDGES, HIDDEN), jnp.bfloat16),
        mesh=plsc.VectorSubcoreMesh(**_MESH),
        compiler_params=pltpu.CompilerParams(use_tc_tiling_on_sc=False),
        scratch_types=[
            pltpu.VMEM_SHARED((_S_ACC, _S_COLS), jnp.bfloat16),
            pltpu.VMEM((_S_CHUNK,), jnp.int32),
            pltpu.VMEM((16, _S_CHUNK), jnp.int32),
            pltpu.VMEM((_S_CHUNK, _S_COLS), jnp.bfloat16),
            pltpu.VMEM((_S_ZROWS, _S_COLS), jnp.bfloat16),
        ],
    )
    def k(u_hbm, i_hbm, o_hbm, acc, raw, lidx, buf, zbuf):
        c = lax.axis_index("c")
        s = lax.axis_index("s")
        base = c * _S_ROWS
        t_base = s * _S_PER

        @pl.loop(0, 16)
        def _(ci):
            @pl.when(ci < 15)
            def _():
                pltpu.sync_copy(
                    i_hbm.at[pl.ds(t_base + ci * _S_CHUNK, _S_CHUNK)], raw)

            @pl.when(ci == 15)
            def _():
                pltpu.sync_copy(
                    i_hbm.at[pl.ds(t_base + 15 * _S_CHUNK, _S_TAIL)],
                    raw.at[pl.ds(0, _S_TAIL)])

            @pl.loop(0, _S_CHUNK, step=16)
            def _(o):
                pos = ci * _S_CHUNK + o + lax.broadcasted_iota(jnp.int32, (16,), 0)
                v = raw[pl.ds(o, 16)] - base
                ok = (v >= 0) & (v < _S_ROWS) & (pos < _S_PER)
                lidx[ci, pl.ds(o, 16)] = jnp.where(ok, v, _S_ROWS)

        @pl.loop(0, _S_ZROWS)
        def _(r):
            zbuf[r, :] = jnp.zeros((_S_COLS,), jnp.bfloat16)

        @pl.loop(0, HIDDEN // _S_COLS)
        def _(p):
            r0 = s * _S_STRIPE

            @pl.loop(0, 16)
            def _(kz):
                pltpu.sync_copy(zbuf, acc.at[pl.ds(r0 + kz * _S_ZROWS, _S_ZROWS)])

            plsc.subcore_barrier()

            @pl.loop(0, 15)
            def _(ci):
                pltpu.sync_copy(
                    u_hbm.at[
                        pl.ds(t_base + ci * _S_CHUNK, _S_CHUNK),
                        pl.ds(p * _S_COLS, _S_COLS),
                    ],
                    buf,
                )
                pltpu.sync_copy(buf, acc.at[lidx.at[ci]], add=True)

            pltpu.sync_copy(
                u_hbm.at[pl.ds(t_base + 15 * _S_CHUNK, _S_TAIL),
                         pl.ds(p * _S_COLS, _S_COLS)],
                buf.at[pl.ds(0, _S_TAIL)],
            )
            pltpu.sync_copy(buf, acc.at[lidx.at[15]], add=True)

            plsc.subcore_barrier()

            @pl.when(s < 15)
            def _():
                pltpu.sync_copy(
                    acc.at[pl.ds(r0, _S_STRIPE)],
                    o_hbm.at[pl.ds(base + r0, _S_STRIPE),
                             pl.ds(p * _S_COLS, _S_COLS)],
                )

            @pl.when(s == 15)
            def _():
                pltpu.sync_copy(
                    acc.at[pl.ds(r0, _S_ROWS - 15 * _S_STRIPE)],
                    o_hbm.at[
                        pl.ds(base + r0, _S_ROWS - 15 * _S_STRIPE),
                        pl.ds(p * _S_COLS, _S_COLS),
                    ],
                )

            plsc.subcore_barrier()

    return k(u, idx)


_A_EDGES = 80000
_A_CHUNK = 128
_A_NCH = 40
_A_PER = 5000
_A_TAIL = _A_PER - 39 * _A_CHUNK
_A_STRIPE = 632
_A_ACC = 10112
_A_DUMP = N_NODES
_A_ZROWS = 79


def _sc_atom_agg(msg, eids, aids):
    @functools.partial(
        pl.kernel,
        out_type=jax.ShapeDtypeStruct((2, N_NODES, HIDDEN), jnp.float32),
        mesh=plsc.VectorSubcoreMesh(**_MESH),
        scratch_types=[
            pltpu.VMEM_SHARED((_A_ACC, HIDDEN), jnp.float32),
            pltpu.VMEM((_A_CHUNK,), jnp.int32),
            pltpu.VMEM((_A_NCH, _A_CHUNK), jnp.int32),
            pltpu.VMEM((_A_NCH, _A_CHUNK), jnp.int32),
            pltpu.VMEM((_A_CHUNK, HIDDEN), jnp.float32),
            pltpu.VMEM((_A_ZROWS, HIDDEN), jnp.float32),
        ],
    )
    def k(m_hbm, e_hbm, a_hbm, o_hbm, acc, raw, eidx, aidx, buf, zbuf):
        c = lax.axis_index("c")
        s = lax.axis_index("s")
        e_base = c * _A_EDGES + s * _A_PER

        @pl.loop(0, _A_NCH)
        def _(ci):
            @pl.when(ci < _A_NCH - 1)
            def _():
                pltpu.sync_copy(e_hbm.at[pl.ds(e_base + ci * _A_CHUNK, _A_CHUNK)], raw)

            @pl.when(ci == _A_NCH - 1)
            def _():
                pltpu.sync_copy(
                    e_hbm.at[pl.ds(e_base + 39 * _A_CHUNK, _A_TAIL)],
                    raw.at[pl.ds(0, _A_TAIL)])

            @pl.loop(0, _A_CHUNK, step=16)
            def _(o):
                pos = ci * _A_CHUNK + o + lax.broadcasted_iota(jnp.int32, (16,), 0)
                eidx[ci, pl.ds(o, 16)] = jnp.where(pos < _A_PER, raw[pl.ds(o, 16)], 0)

            @pl.when(ci < _A_NCH - 1)
            def _():
                pltpu.sync_copy(a_hbm.at[pl.ds(e_base + ci * _A_CHUNK, _A_CHUNK)], raw)

            @pl.when(ci == _A_NCH - 1)
            def _():
                pltpu.sync_copy(
                    a_hbm.at[pl.ds(e_base + 39 * _A_CHUNK, _A_TAIL)],
                    raw.at[pl.ds(0, _A_TAIL)])

            @pl.loop(0, _A_CHUNK, step=16)
            def _(o):
                pos = ci * _A_CHUNK + o + lax.broadcasted_iota(jnp.int32, (16,), 0)
                aidx[ci, pl.ds(o, 16)] = jnp.where(
                    pos < _A_PER, raw[pl.ds(o, 16)], _A_DUMP)

        @pl.loop(0, _A_ZROWS)
        def _(r):
            @pl.loop(0, HIDDEN, step=16)
            def _(o):
                zbuf[r, pl.ds(o, 16)] = jnp.zeros((16,), jnp.float32)

        r0 = s * _A_STRIPE

        @pl.loop(0, 8)
        def _(kz):
            pltpu.sync_copy(zbuf, acc.at[pl.ds(r0 + kz * _A_ZROWS, _A_ZROWS)])

        plsc.subcore_barrier()

        @pl.loop(0, _A_NCH)
        def _(ci):
            pltpu.sync_copy(m_hbm.at[eidx.at[ci]], buf)
            pltpu.sync_copy(buf, acc.at[aidx.at[ci]], add=True)

        plsc.subcore_barrier()

        @pl.when(s < 15)
        def _():
            pltpu.sync_copy(
                acc.at[pl.ds(r0, _A_STRIPE)],
                o_hbm.at[c].at[pl.ds(r0, _A_STRIPE), :],
            )

        @pl.when(s == 15)
        def _():
            pltpu.sync_copy(
                acc.at[pl.ds(r0, N_NODES - 15 * _A_STRIPE)],
                o_hbm.at[c].at[pl.ds(r0, N_NODES - 15 * _A_STRIPE), :],
            )

    return k(msg, eids, aids)


_act = jax.nn.relu


def _lin(x, w, b=None):
    y = jnp.dot(x.astype(jnp.bfloat16), w.astype(jnp.bfloat16),
                preferred_element_type=jnp.float32)
    if b is not None:
        y = y + b
    return y


def _tc_call(body, grid, out_shapes, ins, in_specs, out_specs):
    return pl.pallas_call(
        body,
        grid=grid,
        in_specs=in_specs,
        out_specs=out_specs,
        out_shape=out_shapes,
    )(*ins)


def _full(shape):
    return pl.BlockSpec(shape, lambda g: tuple(0 for _ in shape))


def _rows(bs, w):
    return pl.BlockSpec((bs, w), lambda g: (g, 0))


_BN = 1000
_BE = 1600
_BT = 1600


def _node_tables(af, p):
    w_i1a = p["W_i1_w"][:133]
    emb = p["emb_atom"]
    wl1 = p["emb_lin_w"][0:128]
    wl2 = p["emb_lin_w"][128:256]

    def body(af_r, emb_r, wa_r, w1_r, w2_r, tab_a_r, tab_b2_r, ti_r):
        x_at = _lin(af_r[:, 0:100], emb_r[...])
        tab_a_r[...] = _lin(af_r[...], wa_r[...])
        tab_b2_r[...] = _lin(x_at, w2_r[...])
        ti_r[...] = _lin(x_at, w1_r[...])

    return _tc_call(
        body,
        (N_NODES // _BN,),
        (
            jax.ShapeDtypeStruct((N_NODES, HIDDEN), jnp.float32),
            jax.ShapeDtypeStruct((N_NODES, HIDDEN), jnp.float32),
            jax.ShapeDtypeStruct((N_NODES, HIDDEN), jnp.float32),
        ),
        (af, emb, w_i1a, wl1, wl2),
        [_rows(_BN, 133), _full((100, 128)), _full((133, 128)),
         _full((128, 128)), _full((128, 128))],
        [_rows(_BN, 128), _rows(_BN, 128), _rows(_BN, 128)],
    )


def _edge_stage1(gab, gti, ef, dist2, p, lp):
    w1b = p["W_i1_w"][133:147]
    b1 = p["W_i1_b"].reshape(1, HIDDEN)
    freq = p["bessel_freq"].reshape(1, NUM_RADIAL)
    wr = p["emb_lin_rbf_w"]
    br = p["emb_lin_rbf_b"].reshape(1, HIDDEN)
    w3 = p["emb_lin_w"][256:384]
    be = p["emb_lin_b"].reshape(1, HIDDEN)

    def body(ga_r, gb2_r, gti_r, ef_r, d_r, w1b_r, b1_r, f_r, wr_r, br_r,
             w3_r, be_r, kj_w, kj_b, r2_w, r2_b, dn_w, dn_b,
             msg_r, rbfe_r, xk_r):
        d = d_r[...]
        msg = _act(ga_r[...] + _lin(ef_r[...], w1b_r[...]) + b1_r[...])
        msg_r[...] = msg
        rbf = np.sqrt(2.0 / CUTOFF) * jnp.sin(f_r[...] * d * (1.0 / CUTOFF)) / d
        rbf_h = _act(_lin(rbf, wr_r[...], br_r[...]))
        rbfe = _act(gti_r[...] + gb2_r[...] + _lin(rbf_h, w3_r[...]) + be_r[...])
        rbfe_r[...] = rbfe
        xkj = _act(_lin(msg, kj_w[...], kj_b[...]))
        xkj = xkj * _act(_lin(rbfe, r2_w[...], r2_b[...]))
        xk_r[...] = _act(_lin(xkj, dn_w[...], dn_b[...]))

    nb = N_EDGES // _BE
    return _tc_call(
        body,
        (nb,),
        (
            jax.ShapeDtypeStruct((N_EDGES, HIDDEN), jnp.float32),
            jax.ShapeDtypeStruct((N_EDGES, HIDDEN), jnp.float32),
            jax.ShapeDtypeStruct((N_EDGES, HIDDEN), jnp.float32),
        ),
        (gab, gab, gti, ef, dist2, w1b, b1, freq, wr, br, w3, be,
         lp["lin_kj_w"], lp["lin_kj_b"].reshape(1, HIDDEN),
         lp["lin_rbf2_w"], lp["lin_rbf2_b"].reshape(1, HIDDEN),
         lp["down_w"], lp["down_b"].reshape(1, HIDDEN)),
        [_rows(_BE, 128),
         pl.BlockSpec((_BE, 128), lambda g, _n=nb: (g + _n, 0)),
         _rows(_BE, 128), _rows(_BE, 14), _rows(_BE, 1),
         _full((14, 128)), _full((1, 128)), _full((1, 16)), _full((16, 128)),
         _full((1, 128)), _full((128, 128)), _full((1, 128)),
         _full((128, 128)), _full((1, 128)), _full((128, 128)),
         _full((1, 128)), _full((128, 128)), _full((1, 128))],
        [_rows(_BE, 128), _rows(_BE, 128), _rows(_BE, 128)],
    )


_Z = np.pi * (np.arange(1, NUM_RADIAL + 1)[None, :]
              + 0.5 * np.arange(NUM_SPHERICAL)[:, None])


def _sbf_from(dt, ang, z):
    B = dt.shape[0]
    th16 = jnp.broadcast_to(dt * (np.pi / 2.0), (B, 16))
    s1 = jnp.sin(th16)
    c1 = jnp.cos(th16)
    t = jnp.cos(jnp.broadcast_to(ang, (B, 16)))[:, 0:1]
    col = lax.broadcasted_iota(jnp.int32, (B, 16), 1)
    ar = jnp.ones_like(s1)
    ai = jnp.zeros_like(s1)
    pr = c1 * c1 - s1 * s1
    pi_ = 2.0 * s1 * c1
    for bit in range(4):
        sel = ((col >> bit) & 1) == 1
        nr = ar * pr - ai * pi_
        ni = ar * pi_ + ai * pr
        ar = jnp.where(sel, nr, ar)
        ai = jnp.where(sel, ni, ai)
        if bit < 3:
            pr, pi_ = pr * pr - pi_ * pi_, 2.0 * pr * pi_
    k2 = col + 2
    br = jnp.ones_like(s1)
    bi = jnp.zeros_like(s1)
    qr, qi = c1, s1
    for bit in range(3):
        sel = ((k2 >> bit) & 1) == 1
        nr = br * qr - bi * qi
        ni = br * qi + bi * qr
        br = jnp.where(sel, nr, br)
        bi = jnp.where(sel, ni, bi)
        if bit < 2:
            qr, qi = qr * qr - qi * qi, 2.0 * qr * qi
    srt = jnp.concatenate([ai] * 6, axis=1)
    crt = jnp.concatenate([ar] * 6, axis=1)
    slt = jnp.concatenate(
        [jnp.broadcast_to(bi[:, l:l + 1], (B, 16)) for l in range(6)], axis=1)
    clt = jnp.concatenate(
        [jnp.broadcast_to(br[:, l:l + 1], (B, 16)) for l in range(6)], axis=1)
    s = srt * clt + crt * slt
    cx = crt * clt - srt * slt
    x = dt * z
    inv = 1.0 / x
    j0 = s * inv
    j1 = (s * inv - cx) * inv
    ps = [jnp.ones_like(t), t]
    for l in range(1, NUM_SPHERICAL - 1):
        ps.append(((2.0 * l + 1.0) * t * ps[l] - l * ps[l - 1]) / (l + 1.0))
    parts = [j0[:, 0:16] * ps[0], j1[:, 16:32] * ps[1]]
    jm1, jc = j0, j1
    for ll in range(1, NUM_SPHERICAL - 1):
        jn = (2.0 * ll + 1.0) * inv * jc - jm1
        jm1, jc = jc, jn
        parts.append(jc[:, 16 * (ll + 1):16 * (ll + 2)] * ps[ll + 1])
    return jnp.concatenate(parts, axis=1)


def _tri_stage1(g1, gd, ang2, lp):
    def body(g_r, gd_r, a_r, z_r, s1_w, s2_w, up_w, up_b, u_r, sbf_r):
        dt = gd_r[...] * (1.0 / CUTOFF)
        sbf = _sbf_from(dt, a_r[...], z_r[...])
        sbf_r[...] = sbf.astype(jnp.bfloat16)
        sh = _act(_lin(_act(_lin(sbf, s1_w[...])), s2_w[...]))
        u_r[...] = _act(
            _lin(g_r[...].astype(jnp.float32) * sh, up_w[...], up_b[...])
        ).astype(jnp.bfloat16)

    return _tc_call(
        body,
        (N_TRIPLETS // _BT,),
        (
            jax.ShapeDtypeStruct((N_TRIPLETS, HIDDEN), jnp.bfloat16),
            jax.ShapeDtypeStruct((N_TRIPLETS, 96), jnp.bfloat16),
        ),
        (g1, gd, ang2, jnp.asarray(_Z.reshape(1, -1), jnp.float32),
         lp["lin_sbf1_w"], lp["lin_sbf2_w"], lp["up_w"],
         lp["up_b"].reshape(1, HIDDEN)),
        [_rows(_BT, 128), _rows(_BT, 1), _rows(_BT, 1), _full((1, 96)),
         _full((96, 128)), _full((128, 128)),
         _full((128, 128)), _full((1, 128))],
        [_rows(_BT, 128), _rows(_BT, 96)],
    )


def _tri_stage2(g2, sbf, lp):
    def body(g_r, sbf_r, s1_w, s2_w, up_w, up_b, u_r):
        sh = _act(_lin(_act(_lin(sbf_r[...], s1_w[...])), s2_w[...]))
        u_r[...] = _act(
            _lin(g_r[...].astype(jnp.float32) * sh, up_w[...], up_b[...])
        ).astype(jnp.bfloat16)

    return _tc_call(
        body,
        (N_TRIPLETS // _BT,),
        jax.ShapeDtypeStruct((N_TRIPLETS, HIDDEN), jnp.bfloat16),
        (g2, sbf, lp["lin_sbf1_w"], lp["lin_sbf2_w"], lp["up_w"],
         lp["up_b"].reshape(1, HIDDEN)),
        [_rows(_BT, 128), _rows(_BT, 96), _full((96, 128)), _full((128, 128)),
         _full((128, 128)), _full((1, 128))],
        _rows(_BT, 128),
    )


def _edge_stage2(agg1, msg0, rbfe, lp1, lp2):
    def body(ag_r, m0_r, re_r, r1_w, r1_b, r2_w, r2_b, kj_w, kj_b, rb_w,
             rb_b, dn_w, dn_b, m1_r, xk_r):
        ag = ag_r[...].astype(jnp.float32)
        m = ag + _act(_lin(_act(_lin(ag, r1_w[...], r1_b[...])), r2_w[...], r2_b[...]))
        m1 = m + m0_r[...]
        m1_r[...] = m1
        xkj = _act(_lin(m1, kj_w[...], kj_b[...]))
        xkj = xkj * _act(_lin(re_r[...], rb_w[...], rb_b[...]))
        xk_r[...] = _act(_lin(xkj, dn_w[...], dn_b[...]))

    return _tc_call(
        body,
        (N_EDGES // _BE,),
        (
            jax.ShapeDtypeStruct((N_EDGES, HIDDEN), jnp.float32),
            jax.ShapeDtypeStruct((N_EDGES, HIDDEN), jnp.float32),
        ),
        (agg1, msg0, rbfe,
         lp1["res1_w"], lp1["res1_b"].reshape(1, HIDDEN),
         lp1["res2_w"], lp1["res2_b"].reshape(1, HIDDEN),
         lp2["lin_kj_w"], lp2["lin_kj_b"].reshape(1, HIDDEN),
         lp2["lin_rbf2_w"], lp2["lin_rbf2_b"].reshape(1, HIDDEN),
         lp2["down_w"], lp2["down_b"].reshape(1, HIDDEN)),
        [_rows(_BE, 128)] * 3
        + [_full((128, 128)), _full((1, 128))] * 5,
        [_rows(_BE, 128), _rows(_BE, 128)],
    )


def _edge_stage3(agg2, msg1, lp2):
    def body(ag_r, m1_r, r1_w, r1_b, r2_w, r2_b, m2_r):
        ag = ag_r[...].astype(jnp.float32)
        m = ag + _act(_lin(_act(_lin(ag, r1_w[...], r1_b[...])), r2_w[...], r2_b[...]))
        m2_r[...] = m + m1_r[...]

    return _tc_call(
        body,
        (N_EDGES // _BE,),
        jax.ShapeDtypeStruct((N_EDGES, HIDDEN), jnp.float32),
        (agg2, msg1,
         lp2["res1_w"], lp2["res1_b"].reshape(1, HIDDEN),
         lp2["res2_w"], lp2["res2_b"].reshape(1, HIDDEN)),
        [_rows(_BE, 128)] * 2 + [_full((128, 128)), _full((1, 128))] * 2,
        _rows(_BE, 128),
    )


def _node_out(af, parts, p):
    wo_a = p["W_o_w"][:133]
    wo_m = p["W_o_w"][133:]
    bo = p["W_o_b"].reshape(1, HIDDEN)

    def body(af_r, pt_r, wa_r, wm_r, bo_r, o_r):
        am = pt_r[0] + pt_r[1]
        o_r[...] = _act(_lin(af_r[...], wa_r[...]) + _lin(am, wm_r[...]) + bo_r[...])

    return _tc_call(
        body,
        (N_NODES // _BN,),
        jax.ShapeDtypeStruct((N_NODES, HIDDEN), jnp.float32),
        (af, parts, wo_a, wo_m, bo),
        [_rows(_BN, 133),
         pl.BlockSpec((2, _BN, 128), lambda g: (0, g, 0)),
         _full((133, 128)), _full((128, 128)), _full((1, 128))],
        _rows(_BN, 128),
    )


def kernel(atom_feature, edge_feature, dist, angle, i, j, idx_kj, idx_ji,
           incomebond_edge_ids, incomebond_index_to_atom, params):
    p = params
    lp1, lp2 = p["layers"][0], p["layers"][1]
    dist2 = dist.reshape(N_EDGES, 1)
    ang2 = angle.reshape(N_TRIPLETS, 1)

    t_a, t_b2, t_i = _node_tables(atom_feature, p)
    tab_j = jnp.concatenate([t_a, t_b2], axis=0)
    jj = jnp.concatenate([j, j + N_NODES])
    gab = _sc_gather(tab_j, jj, 128, 2 * N_EDGES)
    g_ti = _sc_gather(t_i, i, 128, N_EDGES)
    msg0, rbfe, xk1 = _edge_stage1(gab, g_ti, edge_feature, dist2, p, lp1)
    gd = _sc_dist_gather(dist, idx_kj).reshape(N_TRIPLETS, 1)
    g1 = _sc_gather(xk1, idx_kj, 128, N_TRIPLETS)
    u1, sbf = _tri_stage1(g1, gd, ang2, lp1)
    agg1 = _sc_scatter_add(u1, idx_kj)
    msg1, xk2 = _edge_stage2(agg1, msg0, rbfe, lp1, lp2)
    g2 = _sc_gather(xk2, idx_kj, 128, N_TRIPLETS)
    u2 = _tri_stage2(g2, sbf, lp2)
    agg2 = _sc_scatter_add(u2, idx_kj)
    msg2 = _edge_stage3(agg2, msg1, lp2)
    parts = _sc_atom_agg(msg2, incomebond_edge_ids, incomebond_index_to_atom)
    return _node_out(atom_feature, parts, p)


# speedup vs baseline: 1.1798x; 1.1798x over previous
import dataclasses
import functools

import jax
import jax.numpy as jnp
import numpy as np
from jax import lax
from jax.experimental import pallas as pl
from jax.experimental.pallas import tpu as pltpu
from jax.experimental.pallas import tpu_sc as plsc

CUTOFF = 8.0
N_NODES = 10000
N_EDGES = 160000
N_TRIPLETS = 320000
HIDDEN = 128
NUM_RADIAL = 16
NUM_SPHERICAL = 6

_MESH = dict(core_axis_name="c", subcore_axis_name="s")


def _sc_gather(table, idx, D, N, W=128):
    idx2 = idx.reshape(1, N)

    @functools.partial(
        pl.kernel,
        out_type=jax.ShapeDtypeStruct((N, D), table.dtype),
        mesh=plsc.VectorSubcoreMesh(**_MESH),
    )
    def k(tab_hbm, i_hbm, o_hbm):
        def body(i_vmem, o_vmem):
            pltpu.sync_copy(tab_hbm.at[i_vmem.at[0]], o_vmem)

        pltpu.emit_pipeline(
            body,
            grid=(N // W,),
            in_specs=[pl.BlockSpec((1, W), lambda g: (0, g))],
            out_specs=[pl.BlockSpec((W, D), lambda g: (g, 0))],
            core_axis_name=("c", "s"),
            dimension_semantics=(pltpu.PARALLEL,),
        )(i_hbm, o_hbm)

    return k(table, idx2)


_D_HALF = 80000
_D_PER = 10000


def _sc_dist_gather(dist, idx):
    cp = pltpu.CompilerParams()
    if "needs_layout_passes" in pltpu.CompilerParams.__dataclass_fields__:
        cp = dataclasses.replace(cp, needs_layout_passes=False)

    @functools.partial(
        pl.kernel,
        out_type=jax.ShapeDtypeStruct((N_TRIPLETS,), jnp.float32),
        mesh=plsc.VectorSubcoreMesh(**_MESH),
        compiler_params=cp,
        scratch_types=[
            pltpu.VMEM((_D_HALF,), jnp.float32),
            pltpu.VMEM((_D_PER,), jnp.int32),
            pltpu.VMEM((_D_PER,), jnp.float32),
        ],
    )
    def k(d_hbm, i_hbm, o_hbm, tab, idxb, outb):
        w = lax.axis_index("c") * 16 + lax.axis_index("s")
        base = w * _D_PER
        pltpu.sync_copy(i_hbm.at[pl.ds(base, _D_PER)], idxb)
        for h in range(2):
            pltpu.sync_copy(d_hbm.at[pl.ds(h * _D_HALF, _D_HALF)], tab)

            @pl.loop(0, _D_PER, step=16)
            def _(o):
                v = idxb[pl.ds(o, 16)] - h * _D_HALF
                m = (v >= 0) & (v < _D_HALF)
                vc = jnp.clip(v, 0, _D_HALF - 1)
                g = plsc.load_gather(tab, [vc])
                r = jnp.where(m, g, 0.0)
                if h == 0:
                    outb[pl.ds(o, 16)] = r
                else:
                    outb[pl.ds(o, 16)] = outb[pl.ds(o, 16)] + r

        pltpu.sync_copy(outb, o_hbm.at[pl.ds(base, _D_PER)])

    return k(dist, idx)


_S_ROWS = 80000
_S_STRIPE = 5008
_S_ACC = 80128
_S_CHUNK = 1280
_S_PER = 20000
_S_TAIL = _S_PER - 15 * _S_CHUNK
_S_COLS = 32
_S_ZROWS = 313


def _sc_scatter_add(u, idx):
    @functools.partial(
        pl.kernel,
        out_type=jax.ShapeDtypeStruct((N_EDGES, HIDDEN), jnp.bfloat16),
        mesh=plsc.VectorSubcoreMesh(**_MESH),
        compiler_params=pltpu.CompilerParams(use_tc_tiling_on_sc=False),
        scratch_types=[
            pltpu.VMEM_SHARED((_S_ACC, _S_COLS), jnp.bfloat16),
            pltpu.VMEM((_S_CHUNK,), jnp.int32),
            pltpu.VMEM((16, _S_CHUNK), jnp.int32),
            pltpu.VMEM((_S_CHUNK, _S_COLS), jnp.bfloat16),
            pltpu.VMEM((_S_ZROWS, _S_COLS), jnp.bfloat16),
        ],
    )
    def k(u_hbm, i_hbm, o_hbm, acc, raw, lidx, buf, zbuf):
        c = lax.axis_index("c")
        s = lax.axis_index("s")
        base = c * _S_ROWS
        t_base = s * _S_PER

        @pl.loop(0, 16)
        def _(ci):
            @pl.when(ci < 15)
            def _():
                pltpu.sync_copy(
                    i_hbm.at[pl.ds(t_base + ci * _S_CHUNK, _S_CHUNK)], raw)

            @pl.when(ci == 15)
            def _():
                pltpu.sync_copy(
                    i_hbm.at[pl.ds(t_base + 15 * _S_CHUNK, _S_TAIL)],
                    raw.at[pl.ds(0, _S_TAIL)])

            @pl.loop(0, _S_CHUNK, step=16)
            def _(o):
                pos = ci * _S_CHUNK + o + lax.broadcasted_iota(jnp.int32, (16,), 0)
                v = raw[pl.ds(o, 16)] - base
                ok = (v >= 0) & (v < _S_ROWS) & (pos < _S_PER)
                lidx[ci, pl.ds(o, 16)] = jnp.where(ok, v, _S_ROWS)

        @pl.loop(0, _S_ZROWS)
        def _(r):
            zbuf[r, :] = jnp.zeros((_S_COLS,), jnp.bfloat16)

        @pl.loop(0, HIDDEN // _S_COLS)
        def _(p):
            r0 = s * _S_STRIPE

            @pl.loop(0, 16)
            def _(kz):
                pltpu.sync_copy(zbuf, acc.at[pl.ds(r0 + kz * _S_ZROWS, _S_ZROWS)])

            plsc.subcore_barrier()

            @pl.loop(0, 15)
            def _(ci):
                pltpu.sync_copy(
                    u_hbm.at[
                        pl.ds(t_base + ci * _S_CHUNK, _S_CHUNK),
                        pl.ds(p * _S_COLS, _S_COLS),
                    ],
                    buf,
                )
                pltpu.sync_copy(buf, acc.at[lidx.at[ci]], add=True)

            pltpu.sync_copy(
                u_hbm.at[pl.ds(t_base + 15 * _S_CHUNK, _S_TAIL),
                         pl.ds(p * _S_COLS, _S_COLS)],
                buf.at[pl.ds(0, _S_TAIL)],
            )
            pltpu.sync_copy(buf, acc.at[lidx.at[15]], add=True)

            plsc.subcore_barrier()

            @pl.when(s < 15)
            def _():
                pltpu.sync_copy(
                    acc.at[pl.ds(r0, _S_STRIPE)],
                    o_hbm.at[pl.ds(base + r0, _S_STRIPE),
                             pl.ds(p * _S_COLS, _S_COLS)],
                )

            @pl.when(s == 15)
            def _():
                pltpu.sync_copy(
                    acc.at[pl.ds(r0, _S_ROWS - 15 * _S_STRIPE)],
                    o_hbm.at[
                        pl.ds(base + r0, _S_ROWS - 15 * _S_STRIPE),
                        pl.ds(p * _S_COLS, _S_COLS),
                    ],
                )

            plsc.subcore_barrier()

    return k(u, idx)


_A_EDGES = 80000
_A_CHUNK = 128
_A_NCH = 40
_A_PER = 5000
_A_TAIL = _A_PER - 39 * _A_CHUNK
_A_STRIPE = 632
_A_ACC = 10112
_A_DUMP = N_NODES
_A_ZROWS = 79


def _sc_atom_agg(msg, eids, aids):
    @functools.partial(
        pl.kernel,
        out_type=jax.ShapeDtypeStruct((2, N_NODES, HIDDEN), jnp.float32),
        mesh=plsc.VectorSubcoreMesh(**_MESH),
        scratch_types=[
            pltpu.VMEM_SHARED((_A_ACC, HIDDEN), jnp.float32),
            pltpu.VMEM((_A_CHUNK,), jnp.int32),
            pltpu.VMEM((_A_NCH, _A_CHUNK), jnp.int32),
            pltpu.VMEM((_A_NCH, _A_CHUNK), jnp.int32),
            pltpu.VMEM((_A_CHUNK, HIDDEN), jnp.float32),
            pltpu.VMEM((_A_ZROWS, HIDDEN), jnp.float32),
        ],
    )
    def k(m_hbm, e_hbm, a_hbm, o_hbm, acc, raw, eidx, aidx, buf, zbuf):
        c = lax.axis_index("c")
        s = lax.axis_index("s")
        e_base = c * _A_EDGES + s * _A_PER

        @pl.loop(0, _A_NCH)
        def _(ci):
            @pl.when(ci < _A_NCH - 1)
            def _():
                pltpu.sync_copy(e_hbm.at[pl.ds(e_base + ci * _A_CHUNK, _A_CHUNK)], raw)

            @pl.when(ci == _A_NCH - 1)
            def _():
                pltpu.sync_copy(
                    e_hbm.at[pl.ds(e_base + 39 * _A_CHUNK, _A_TAIL)],
                    raw.at[pl.ds(0, _A_TAIL)])

            @pl.loop(0, _A_CHUNK, step=16)
            def _(o):
                pos = ci * _A_CHUNK + o + lax.broadcasted_iota(jnp.int32, (16,), 0)
                eidx[ci, pl.ds(o, 16)] = jnp.where(pos < _A_PER, raw[pl.ds(o, 16)], 0)

            @pl.when(ci < _A_NCH - 1)
            def _():
                pltpu.sync_copy(a_hbm.at[pl.ds(e_base + ci * _A_CHUNK, _A_CHUNK)], raw)

            @pl.when(ci == _A_NCH - 1)
            def _():
                pltpu.sync_copy(
                    a_hbm.at[pl.ds(e_base + 39 * _A_CHUNK, _A_TAIL)],
                    raw.at[pl.ds(0, _A_TAIL)])

            @pl.loop(0, _A_CHUNK, step=16)
            def _(o):
                pos = ci * _A_CHUNK + o + lax.broadcasted_iota(jnp.int32, (16,), 0)
                aidx[ci, pl.ds(o, 16)] = jnp.where(
                    pos < _A_PER, raw[pl.ds(o, 16)], _A_DUMP)

        @pl.loop(0, _A_ZROWS)
        def _(r):
            @pl.loop(0, HIDDEN, step=16)
            def _(o):
                zbuf[r, pl.ds(o, 16)] = jnp.zeros((16,), jnp.float32)

        r0 = s * _A_STRIPE

        @pl.loop(0, 8)
        def _(kz):
            pltpu.sync_copy(zbuf, acc.at[pl.ds(r0 + kz * _A_ZROWS, _A_ZROWS)])

        plsc.subcore_barrier()

        @pl.loop(0, _A_NCH)
        def _(ci):
            pltpu.sync_copy(m_hbm.at[eidx.at[ci]], buf)
            pltpu.sync_copy(buf, acc.at[aidx.at[ci]], add=True)

        plsc.subcore_barrier()

        @pl.when(s < 15)
        def _():
            pltpu.sync_copy(
                acc.at[pl.ds(r0, _A_STRIPE)],
                o_hbm.at[c].at[pl.ds(r0, _A_STRIPE), :],
            )

        @pl.when(s == 15)
        def _():
            pltpu.sync_copy(
                acc.at[pl.ds(r0, N_NODES - 15 * _A_STRIPE)],
                o_hbm.at[c].at[pl.ds(r0, N_NODES - 15 * _A_STRIPE), :],
            )

    return k(msg, eids, aids)


_act = jax.nn.relu


def _lin(x, w, b=None):
    y = jnp.dot(x.astype(jnp.bfloat16), w.astype(jnp.bfloat16),
                preferred_element_type=jnp.float32)
    if b is not None:
        y = y + b
    return y


def _tc_call(body, grid, out_shapes, ins, in_specs, out_specs):
    return pl.pallas_call(
        body,
        grid=grid,
        in_specs=in_specs,
        out_specs=out_specs,
        out_shape=out_shapes,
    )(*ins)


def _full(shape):
    return pl.BlockSpec(shape, lambda g: tuple(0 for _ in shape))


def _rows(bs, w):
    return pl.BlockSpec((bs, w), lambda g: (g, 0))


_BN = 1000
_BE = 1600
_BT = 1600


def _node_tables(af, p):
    w_i1a = p["W_i1_w"][:133]
    emb = p["emb_atom"]
    wl1 = p["emb_lin_w"][0:128]
    wl2 = p["emb_lin_w"][128:256]

    def body(af_r, emb_r, wa_r, w1_r, w2_r, tab_a_r, tab_b2_r, ti_r):
        x_at = _lin(af_r[:, 0:100], emb_r[...])
        tab_a_r[...] = _lin(af_r[...], wa_r[...])
        tab_b2_r[...] = _lin(x_at, w2_r[...])
        ti_r[...] = _lin(x_at, w1_r[...])

    return _tc_call(
        body,
        (N_NODES // _BN,),
        (
            jax.ShapeDtypeStruct((N_NODES, HIDDEN), jnp.float32),
            jax.ShapeDtypeStruct((N_NODES, HIDDEN), jnp.float32),
            jax.ShapeDtypeStruct((N_NODES, HIDDEN), jnp.float32),
        ),
        (af, emb, w_i1a, wl1, wl2),
        [_rows(_BN, 133), _full((100, 128)), _full((133, 128)),
         _full((128, 128)), _full((128, 128))],
        [_rows(_BN, 128), _rows(_BN, 128), _rows(_BN, 128)],
    )


def _edge_stage1(gab, gti, ef, dist2, p, lp):
    w1b = p["W_i1_w"][133:147]
    b1 = p["W_i1_b"].reshape(1, HIDDEN)
    freq = p["bessel_freq"].reshape(1, NUM_RADIAL)
    wr = p["emb_lin_rbf_w"]
    br = p["emb_lin_rbf_b"].reshape(1, HIDDEN)
    w3 = p["emb_lin_w"][256:384]
    be = p["emb_lin_b"].reshape(1, HIDDEN)

    def body(ga_r, gb2_r, gti_r, ef_r, d_r, w1b_r, b1_r, f_r, wr_r, br_r,
             w3_r, be_r, kj_w, kj_b, r2_w, r2_b, dn_w, dn_b,
             msg_r, rbfe_r, xk_r):
        d = d_r[...]
        msg = _act(ga_r[...] + _lin(ef_r[...], w1b_r[...]) + b1_r[...])
        msg_r[...] = msg
        rbf = np.sqrt(2.0 / CUTOFF) * jnp.sin(f_r[...] * d * (1.0 / CUTOFF)) / d
        rbf_h = _act(_lin(rbf, wr_r[...], br_r[...]))
        rbfe = _act(gti_r[...] + gb2_r[...] + _lin(rbf_h, w3_r[...]) + be_r[...])
        rbfe_r[...] = rbfe
        xkj = _act(_lin(msg, kj_w[...], kj_b[...]))
        xkj = xkj * _act(_lin(rbfe, r2_w[...], r2_b[...]))
        xk_r[...] = _act(_lin(xkj, dn_w[...], dn_b[...]))

    nb = N_EDGES // _BE
    return _tc_call(
        body,
        (nb,),
        (
            jax.ShapeDtypeStruct((N_EDGES, HIDDEN), jnp.float32),
            jax.ShapeDtypeStruct((N_EDGES, HIDDEN), jnp.float32),
            jax.ShapeDtypeStruct((N_EDGES, HIDDEN), jnp.float32),
        ),
        (gab, gab, gti, ef, dist2, w1b, b1, freq, wr, br, w3, be,
         lp["lin_kj_w"], lp["lin_kj_b"].reshape(1, HIDDEN),
         lp["lin_rbf2_w"], lp["lin_rbf2_b"].reshape(1, HIDDEN),
         lp["down_w"], lp["down_b"].reshape(1, HIDDEN)),
        [_rows(_BE, 128),
         pl.BlockSpec((_BE, 128), lambda g, _n=nb: (g + _n, 0)),
         _rows(_BE, 128), _rows(_BE, 14), _rows(_BE, 1),
         _full((14, 128)), _full((1, 128)), _full((1, 16)), _full((16, 128)),
         _full((1, 128)), _full((128, 128)), _full((1, 128)),
         _full((128, 128)), _full((1, 128)), _full((128, 128)),
         _full((1, 128)), _full((128, 128)), _full((1, 128))],
        [_rows(_BE, 128), _rows(_BE, 128), _rows(_BE, 128)],
    )


_Z = np.pi * (np.arange(1, NUM_RADIAL + 1)[None, :]
              + 0.5 * np.arange(NUM_SPHERICAL)[:, None])


def _sbf_from(dt, ang, z):
    B = dt.shape[0]
    t = jnp.cos(jnp.broadcast_to(ang, (B, 16)))[:, 0:1]
    x = dt * z
    s = jnp.sin(x)
    cx = jnp.cos(x)
    inv = 1.0 / x
    j0 = s * inv
    j1 = (s * inv - cx) * inv
    ps = [jnp.ones_like(t), t]
    for l in range(1, NUM_SPHERICAL - 1):
        ps.append(((2.0 * l + 1.0) * t * ps[l] - l * ps[l - 1]) / (l + 1.0))
    parts = [j0[:, 0:16] * ps[0], j1[:, 16:32] * ps[1]]
    jm1, jc = j0, j1
    for ll in range(1, NUM_SPHERICAL - 1):
        jn = (2.0 * ll + 1.0) * inv * jc - jm1
        jm1, jc = jc, jn
        parts.append(jc[:, 16 * (ll + 1):16 * (ll + 2)] * ps[ll + 1])
    return jnp.concatenate(parts, axis=1)


def _tri_stage1(g1, gd, ang2, lp):
    def body(g_r, gd_r, a_r, z_r, s1_w, s2_w, up_w, up_b, u_r, sbf_r):
        dt = gd_r[...] * (1.0 / CUTOFF)
        sbf = _sbf_from(dt, a_r[...], z_r[...])
        sbf_r[...] = sbf.astype(jnp.bfloat16)
        sh = _act(_lin(_act(_lin(sbf, s1_w[...])), s2_w[...]))
        u_r[...] = _act(
            _lin(g_r[...].astype(jnp.float32) * sh, up_w[...], up_b[...])
        ).astype(jnp.bfloat16)

    return _tc_call(
        body,
        (N_TRIPLETS // _BT,),
        (
            jax.ShapeDtypeStruct((N_TRIPLETS, HIDDEN), jnp.bfloat16),
            jax.ShapeDtypeStruct((N_TRIPLETS, 96), jnp.bfloat16),
        ),
        (g1, gd, ang2, jnp.asarray(_Z.reshape(1, -1), jnp.float32),
         lp["lin_sbf1_w"], lp["lin_sbf2_w"], lp["up_w"],
         lp["up_b"].reshape(1, HIDDEN)),
        [_rows(_BT, 128), _rows(_BT, 1), _rows(_BT, 1), _full((1, 96)),
         _full((96, 128)), _full((128, 128)),
         _full((128, 128)), _full((1, 128))],
        [_rows(_BT, 128), _rows(_BT, 96)],
    )


def _tri_stage2(g2, sbf, lp):
    def body(g_r, sbf_r, s1_w, s2_w, up_w, up_b, u_r):
        sh = _act(_lin(_act(_lin(sbf_r[...], s1_w[...])), s2_w[...]))
        u_r[...] = _act(
            _lin(g_r[...].astype(jnp.float32) * sh, up_w[...], up_b[...])
        ).astype(jnp.bfloat16)

    return _tc_call(
        body,
        (N_TRIPLETS // _BT,),
        jax.ShapeDtypeStruct((N_TRIPLETS, HIDDEN), jnp.bfloat16),
        (g2, sbf, lp["lin_sbf1_w"], lp["lin_sbf2_w"], lp["up_w"],
         lp["up_b"].reshape(1, HIDDEN)),
        [_rows(_BT, 128), _rows(_BT, 96), _full((96, 128)), _full((128, 128)),
         _full((128, 128)), _full((1, 128))],
        _rows(_BT, 128),
    )


def _edge_stage2(agg1, msg0, rbfe, lp1, lp2):
    def body(ag_r, m0_r, re_r, r1_w, r1_b, r2_w, r2_b, kj_w, kj_b, rb_w,
             rb_b, dn_w, dn_b, m1_r, xk_r):
        ag = ag_r[...].astype(jnp.float32)
        m = ag + _act(_lin(_act(_lin(ag, r1_w[...], r1_b[...])), r2_w[...], r2_b[...]))
        m1 = m + m0_r[...]
        m1_r[...] = m1
        xkj = _act(_lin(m1, kj_w[...], kj_b[...]))
        xkj = xkj * _act(_lin(re_r[...], rb_w[...], rb_b[...]))
        xk_r[...] = _act(_lin(xkj, dn_w[...], dn_b[...]))

    return _tc_call(
        body,
        (N_EDGES // _BE,),
        (
            jax.ShapeDtypeStruct((N_EDGES, HIDDEN), jnp.float32),
            jax.ShapeDtypeStruct((N_EDGES, HIDDEN), jnp.float32),
        ),
        (agg1, msg0, rbfe,
         lp1["res1_w"], lp1["res1_b"].reshape(1, HIDDEN),
         lp1["res2_w"], lp1["res2_b"].reshape(1, HIDDEN),
         lp2["lin_kj_w"], lp2["lin_kj_b"].reshape(1, HIDDEN),
         lp2["lin_rbf2_w"], lp2["lin_rbf2_b"].reshape(1, HIDDEN),
         lp2["down_w"], lp2["down_b"].reshape(1, HIDDEN)),
        [_rows(_BE, 128)] * 3
        + [_full((128, 128)), _full((1, 128))] * 5,
        [_rows(_BE, 128), _rows(_BE, 128)],
    )


def _edge_stage3(agg2, msg1, lp2):
    def body(ag_r, m1_r, r1_w, r1_b, r2_w, r2_b, m2_r):
        ag = ag_r[...].astype(jnp.float32)
        m = ag + _act(_lin(_act(_lin(ag, r1_w[...], r1_b[...])), r2_w[...], r2_b[...]))
        m2_r[...] = m + m1_r[...]

    return _tc_call(
        body,
        (N_EDGES // _BE,),
        jax.ShapeDtypeStruct((N_EDGES, HIDDEN), jnp.float32),
        (agg2, msg1,
         lp2["res1_w"], lp2["res1_b"].reshape(1, HIDDEN),
         lp2["res2_w"], lp2["res2_b"].reshape(1, HIDDEN)),
        [_rows(_BE, 128)] * 2 + [_full((128, 128)), _full((1, 128))] * 2,
        _rows(_BE, 128),
    )


def _node_out(af, parts, p):
    wo_a = p["W_o_w"][:133]
    wo_m = p["W_o_w"][133:]
    bo = p["W_o_b"].reshape(1, HIDDEN)

    def body(af_r, pt_r, wa_r, wm_r, bo_r, o_r):
        am = pt_r[0] + pt_r[1]
        o_r[...] = _act(_lin(af_r[...], wa_r[...]) + _lin(am, wm_r[...]) + bo_r[...])

    return _tc_call(
        body,
        (N_NODES // _BN,),
        jax.ShapeDtypeStruct((N_NODES, HIDDEN), jnp.float32),
        (af, parts, wo_a, wo_m, bo),
        [_rows(_BN, 133),
         pl.BlockSpec((2, _BN, 128), lambda g: (0, g, 0)),
         _full((133, 128)), _full((128, 128)), _full((1, 128))],
        _rows(_BN, 128),
    )


def kernel(atom_feature, edge_feature, dist, angle, i, j, idx_kj, idx_ji,
           incomebond_edge_ids, incomebond_index_to_atom, params):
    p = params
    lp1, lp2 = p["layers"][0], p["layers"][1]
    dist2 = dist.reshape(N_EDGES, 1)
    ang2 = angle.reshape(N_TRIPLETS, 1)

    t_a, t_b2, t_i = _node_tables(atom_feature, p)
    tab_j = jnp.concatenate([t_a, t_b2], axis=0)
    jj = jnp.concatenate([j, j + N_NODES])
    gab = _sc_gather(tab_j, jj, 128, 2 * N_EDGES)
    g_ti = _sc_gather(t_i, i, 128, N_EDGES)
    msg0, rbfe, xk1 = _edge_stage1(gab, g_ti, edge_feature, dist2, p, lp1)
    gd = _sc_dist_gather(dist, idx_kj).reshape(N_TRIPLETS, 1)
    g1 = _sc_gather(xk1, idx_kj, 128, N_TRIPLETS)
    u1, sbf = _tri_stage1(g1, gd, ang2, lp1)
    agg1 = _sc_scatter_add(u1, idx_kj)
    msg1, xk2 = _edge_stage2(agg1, msg0, rbfe, lp1, lp2)
    g2 = _sc_gather(xk2, idx_kj, 128, N_TRIPLETS)
    u2 = _tri_stage2(g2, sbf, lp2)
    agg2 = _sc_scatter_add(u2, idx_kj)
    msg2 = _edge_stage3(agg2, msg1, lp2)
    parts = _sc_atom_agg(msg2, incomebond_edge_ids, incomebond_index_to_atom)
    return _node_out(atom_feature, parts, p)


# speedup vs baseline: 1.4843x; 1.2581x over previous
import dataclasses
import functools

import jax
import jax.numpy as jnp
import numpy as np
from jax import lax
from jax.experimental import pallas as pl
from jax.experimental.pallas import tpu as pltpu
from jax.experimental.pallas import tpu_sc as plsc

CUTOFF = 8.0
N_NODES = 10000
N_EDGES = 160000
N_TRIPLETS = 320000
HIDDEN = 128
NUM_RADIAL = 16
NUM_SPHERICAL = 6

_MESH = dict(core_axis_name="c", subcore_axis_name="s")


def _sc_gather(table, idx, D, N, W=128):
    idx2 = idx.reshape(1, N)

    @functools.partial(
        pl.kernel,
        out_type=jax.ShapeDtypeStruct((N, D), table.dtype),
        mesh=plsc.VectorSubcoreMesh(**_MESH),
    )
    def k(tab_hbm, i_hbm, o_hbm):
        def body(i_vmem, o_vmem):
            pltpu.sync_copy(tab_hbm.at[i_vmem.at[0]], o_vmem)

        pltpu.emit_pipeline(
            body,
            grid=(N // W,),
            in_specs=[pl.BlockSpec((1, W), lambda g: (0, g))],
            out_specs=[pl.BlockSpec((W, D), lambda g: (g, 0))],
            core_axis_name=("c", "s"),
            dimension_semantics=(pltpu.PARALLEL,),
        )(i_hbm, o_hbm)

    return k(table, idx2)


_D_HALF = 80000
_D_PER = 10000


def _sc_dist_gather(dist, idx):
    cp = pltpu.CompilerParams()
    if "needs_layout_passes" in pltpu.CompilerParams.__dataclass_fields__:
        cp = dataclasses.replace(cp, needs_layout_passes=False)

    @functools.partial(
        pl.kernel,
        out_type=jax.ShapeDtypeStruct((N_TRIPLETS,), jnp.float32),
        mesh=plsc.VectorSubcoreMesh(**_MESH),
        compiler_params=cp,
        scratch_types=[
            pltpu.VMEM((_D_HALF,), jnp.float32),
            pltpu.VMEM((_D_PER,), jnp.int32),
            pltpu.VMEM((_D_PER,), jnp.float32),
        ],
    )
    def k(d_hbm, i_hbm, o_hbm, tab, idxb, outb):
        w = lax.axis_index("c") * 16 + lax.axis_index("s")
        base = w * _D_PER
        pltpu.sync_copy(i_hbm.at[pl.ds(base, _D_PER)], idxb)
        for h in range(2):
            pltpu.sync_copy(d_hbm.at[pl.ds(h * _D_HALF, _D_HALF)], tab)

            @pl.loop(0, _D_PER, step=16)
            def _(o):
                v = idxb[pl.ds(o, 16)] - h * _D_HALF
                m = (v >= 0) & (v < _D_HALF)
                vc = jnp.clip(v, 0, _D_HALF - 1)
                g = plsc.load_gather(tab, [vc])
                r = jnp.where(m, g, 0.0)
                if h == 0:
                    outb[pl.ds(o, 16)] = r
                else:
                    outb[pl.ds(o, 16)] = outb[pl.ds(o, 16)] + r

        pltpu.sync_copy(outb, o_hbm.at[pl.ds(base, _D_PER)])

    return k(dist, idx)


_S_ROWS = 80000
_S_STRIPE = 5008
_S_ACC = 80128
_S_CHUNK = 1280
_S_PER = 20000
_S_TAIL = _S_PER - 15 * _S_CHUNK
_S_COLS = 32
_S_ZROWS = 313


def _sc_scatter_add(u, idx):
    @functools.partial(
        pl.kernel,
        out_type=jax.ShapeDtypeStruct((N_EDGES, HIDDEN), jnp.bfloat16),
        mesh=plsc.VectorSubcoreMesh(**_MESH),
        compiler_params=pltpu.CompilerParams(use_tc_tiling_on_sc=False),
        scratch_types=[
            pltpu.VMEM_SHARED((_S_ACC, _S_COLS), jnp.bfloat16),
            pltpu.VMEM((_S_CHUNK,), jnp.int32),
            pltpu.VMEM((16, _S_CHUNK), jnp.int32),
            pltpu.VMEM((_S_CHUNK, _S_COLS), jnp.bfloat16),
            pltpu.VMEM((_S_ZROWS, _S_COLS), jnp.bfloat16),
        ],
    )
    def k(u_hbm, i_hbm, o_hbm, acc, raw, lidx, buf, zbuf):
        c = lax.axis_index("c")
        s = lax.axis_index("s")
        base = c * _S_ROWS
        t_base = s * _S_PER

        @pl.loop(0, 16)
        def _(ci):
            @pl.when(ci < 15)
            def _():
                pltpu.sync_copy(
                    i_hbm.at[pl.ds(t_base + ci * _S_CHUNK, _S_CHUNK)], raw)

            @pl.when(ci == 15)
            def _():
                pltpu.sync_copy(
                    i_hbm.at[pl.ds(t_base + 15 * _S_CHUNK, _S_TAIL)],
                    raw.at[pl.ds(0, _S_TAIL)])

            @pl.loop(0, _S_CHUNK, step=16)
            def _(o):
                pos = ci * _S_CHUNK + o + lax.broadcasted_iota(jnp.int32, (16,), 0)
                v = raw[pl.ds(o, 16)] - base
                ok = (v >= 0) & (v < _S_ROWS) & (pos < _S_PER)
                lidx[ci, pl.ds(o, 16)] = jnp.where(ok, v, _S_ROWS)

        @pl.loop(0, _S_ZROWS)
        def _(r):
            zbuf[r, :] = jnp.zeros((_S_COLS,), jnp.bfloat16)

        @pl.loop(0, HIDDEN // _S_COLS)
        def _(p):
            r0 = s * _S_STRIPE

            @pl.loop(0, 16)
            def _(kz):
                pltpu.sync_copy(zbuf, acc.at[pl.ds(r0 + kz * _S_ZROWS, _S_ZROWS)])

            plsc.subcore_barrier()

            @pl.loop(0, 15)
            def _(ci):
                pltpu.sync_copy(
                    u_hbm.at[
                        pl.ds(t_base + ci * _S_CHUNK, _S_CHUNK),
                        pl.ds(p * _S_COLS, _S_COLS),
                    ],
                    buf,
                )
                pltpu.sync_copy(buf, acc.at[lidx.at[ci]], add=True)

            pltpu.sync_copy(
                u_hbm.at[pl.ds(t_base + 15 * _S_CHUNK, _S_TAIL),
                         pl.ds(p * _S_COLS, _S_COLS)],
                buf.at[pl.ds(0, _S_TAIL)],
            )
            pltpu.sync_copy(buf, acc.at[lidx.at[15]], add=True)

            plsc.subcore_barrier()

            @pl.when(s < 15)
            def _():
                pltpu.sync_copy(
                    acc.at[pl.ds(r0, _S_STRIPE)],
                    o_hbm.at[pl.ds(base + r0, _S_STRIPE),
                             pl.ds(p * _S_COLS, _S_COLS)],
                )

            @pl.when(s == 15)
            def _():
                pltpu.sync_copy(
                    acc.at[pl.ds(r0, _S_ROWS - 15 * _S_STRIPE)],
                    o_hbm.at[
                        pl.ds(base + r0, _S_ROWS - 15 * _S_STRIPE),
                        pl.ds(p * _S_COLS, _S_COLS),
                    ],
                )

            plsc.subcore_barrier()

    return k(u, idx)


_A_EDGES = 80000
_A_CHUNK = 128
_A_NCH = 40
_A_PER = 5000
_A_TAIL = _A_PER - 39 * _A_CHUNK
_A_STRIPE = 632
_A_ACC = 10112
_A_DUMP = N_NODES
_A_ZROWS = 79


def _sc_atom_agg(msg, eids, aids):
    @functools.partial(
        pl.kernel,
        out_type=jax.ShapeDtypeStruct((2, N_NODES, HIDDEN), jnp.float32),
        mesh=plsc.VectorSubcoreMesh(**_MESH),
        scratch_types=[
            pltpu.VMEM_SHARED((_A_ACC, HIDDEN), jnp.float32),
            pltpu.VMEM((_A_CHUNK,), jnp.int32),
            pltpu.VMEM((_A_NCH, _A_CHUNK), jnp.int32),
            pltpu.VMEM((_A_NCH, _A_CHUNK), jnp.int32),
            pltpu.VMEM((_A_CHUNK, HIDDEN), jnp.float32),
            pltpu.VMEM((_A_ZROWS, HIDDEN), jnp.float32),
        ],
    )
    def k(m_hbm, e_hbm, a_hbm, o_hbm, acc, raw, eidx, aidx, buf, zbuf):
        c = lax.axis_index("c")
        s = lax.axis_index("s")
        e_base = c * _A_EDGES + s * _A_PER

        @pl.loop(0, _A_NCH)
        def _(ci):
            @pl.when(ci < _A_NCH - 1)
            def _():
                pltpu.sync_copy(e_hbm.at[pl.ds(e_base + ci * _A_CHUNK, _A_CHUNK)], raw)

            @pl.when(ci == _A_NCH - 1)
            def _():
                pltpu.sync_copy(
                    e_hbm.at[pl.ds(e_base + 39 * _A_CHUNK, _A_TAIL)],
                    raw.at[pl.ds(0, _A_TAIL)])

            @pl.loop(0, _A_CHUNK, step=16)
            def _(o):
                pos = ci * _A_CHUNK + o + lax.broadcasted_iota(jnp.int32, (16,), 0)
                eidx[ci, pl.ds(o, 16)] = jnp.where(pos < _A_PER, raw[pl.ds(o, 16)], 0)

            @pl.when(ci < _A_NCH - 1)
            def _():
                pltpu.sync_copy(a_hbm.at[pl.ds(e_base + ci * _A_CHUNK, _A_CHUNK)], raw)

            @pl.when(ci == _A_NCH - 1)
            def _():
                pltpu.sync_copy(
                    a_hbm.at[pl.ds(e_base + 39 * _A_CHUNK, _A_TAIL)],
                    raw.at[pl.ds(0, _A_TAIL)])

            @pl.loop(0, _A_CHUNK, step=16)
            def _(o):
                pos = ci * _A_CHUNK + o + lax.broadcasted_iota(jnp.int32, (16,), 0)
                aidx[ci, pl.ds(o, 16)] = jnp.where(
                    pos < _A_PER, raw[pl.ds(o, 16)], _A_DUMP)

        @pl.loop(0, _A_ZROWS)
        def _(r):
            @pl.loop(0, HIDDEN, step=16)
            def _(o):
                zbuf[r, pl.ds(o, 16)] = jnp.zeros((16,), jnp.float32)

        r0 = s * _A_STRIPE

        @pl.loop(0, 8)
        def _(kz):
            pltpu.sync_copy(zbuf, acc.at[pl.ds(r0 + kz * _A_ZROWS, _A_ZROWS)])

        plsc.subcore_barrier()

        @pl.loop(0, _A_NCH)
        def _(ci):
            pltpu.sync_copy(m_hbm.at[eidx.at[ci]], buf)
            pltpu.sync_copy(buf, acc.at[aidx.at[ci]], add=True)

        plsc.subcore_barrier()

        @pl.when(s < 15)
        def _():
            pltpu.sync_copy(
                acc.at[pl.ds(r0, _A_STRIPE)],
                o_hbm.at[c].at[pl.ds(r0, _A_STRIPE), :],
            )

        @pl.when(s == 15)
        def _():
            pltpu.sync_copy(
                acc.at[pl.ds(r0, N_NODES - 15 * _A_STRIPE)],
                o_hbm.at[c].at[pl.ds(r0, N_NODES - 15 * _A_STRIPE), :],
            )

    return k(msg, eids, aids)


_act = jax.nn.relu


def _lin(x, w, b=None):
    y = jnp.dot(x.astype(jnp.bfloat16), w.astype(jnp.bfloat16),
                preferred_element_type=jnp.float32)
    if b is not None:
        y = y + b
    return y


def _tc_call(body, grid, out_shapes, ins, in_specs, out_specs):
    return pl.pallas_call(
        body,
        grid=grid,
        in_specs=in_specs,
        out_specs=out_specs,
        out_shape=out_shapes,
    )(*ins)


def _full(shape):
    return pl.BlockSpec(shape, lambda g: tuple(0 for _ in shape))


def _rows(bs, w):
    return pl.BlockSpec((bs, w), lambda g: (g, 0))


_BN = 1000
_BE = 1600
_BT = 1280


def _node_tables(af, p):
    w_i1a = p["W_i1_w"][:133]
    emb = p["emb_atom"]
    wl1 = p["emb_lin_w"][0:128]
    wl2 = p["emb_lin_w"][128:256]

    def body(af_r, emb_r, wa_r, w1_r, w2_r, tab_a_r, tab_b2_r, ti_r):
        x_at = _lin(af_r[:, 0:100], emb_r[...])
        tab_a_r[...] = _lin(af_r[...], wa_r[...])
        tab_b2_r[...] = _lin(x_at, w2_r[...])
        ti_r[...] = _lin(x_at, w1_r[...])

    return _tc_call(
        body,
        (N_NODES // _BN,),
        (
            jax.ShapeDtypeStruct((N_NODES, HIDDEN), jnp.float32),
            jax.ShapeDtypeStruct((N_NODES, HIDDEN), jnp.float32),
            jax.ShapeDtypeStruct((N_NODES, HIDDEN), jnp.float32),
        ),
        (af, emb, w_i1a, wl1, wl2),
        [_rows(_BN, 133), _full((100, 128)), _full((133, 128)),
         _full((128, 128)), _full((128, 128))],
        [_rows(_BN, 128), _rows(_BN, 128), _rows(_BN, 128)],
    )


def _edge_stage1(gab, gti, ef, dist2, p, lp):
    w1b = p["W_i1_w"][133:147]
    b1 = p["W_i1_b"].reshape(1, HIDDEN)
    freq = p["bessel_freq"].reshape(1, NUM_RADIAL)
    wr = p["emb_lin_rbf_w"]
    br = p["emb_lin_rbf_b"].reshape(1, HIDDEN)
    w3 = p["emb_lin_w"][256:384]
    be = p["emb_lin_b"].reshape(1, HIDDEN)

    def body(ga_r, gb2_r, gti_r, ef_r, d_r, w1b_r, b1_r, f_r, wr_r, br_r,
             w3_r, be_r, kj_w, kj_b, r2_w, r2_b, dn_w, dn_b,
             msg_r, rbfe_r, xk_r):
        d = d_r[...]
        msg = _act(ga_r[...] + _lin(ef_r[...], w1b_r[...]) + b1_r[...])
        msg_r[...] = msg
        rbf = np.sqrt(2.0 / CUTOFF) * jnp.sin(f_r[...] * d * (1.0 / CUTOFF)) / d
        rbf_h = _act(_lin(rbf, wr_r[...], br_r[...]))
        rbfe = _act(gti_r[...] + gb2_r[...] + _lin(rbf_h, w3_r[...]) + be_r[...])
        rbfe_r[...] = rbfe
        xkj = _act(_lin(msg, kj_w[...], kj_b[...]))
        xkj = xkj * _act(_lin(rbfe, r2_w[...], r2_b[...]))
        xk_r[...] = _act(_lin(xkj, dn_w[...], dn_b[...]))

    nb = N_EDGES // _BE
    return _tc_call(
        body,
        (nb,),
        (
            jax.ShapeDtypeStruct((N_EDGES, HIDDEN), jnp.float32),
            jax.ShapeDtypeStruct((N_EDGES, HIDDEN), jnp.float32),
            jax.ShapeDtypeStruct((N_EDGES, HIDDEN), jnp.float32),
        ),
        (gab, gab, gti, ef, dist2, w1b, b1, freq, wr, br, w3, be,
         lp["lin_kj_w"], lp["lin_kj_b"].reshape(1, HIDDEN),
         lp["lin_rbf2_w"], lp["lin_rbf2_b"].reshape(1, HIDDEN),
         lp["down_w"], lp["down_b"].reshape(1, HIDDEN)),
        [_rows(_BE, 128),
         pl.BlockSpec((_BE, 128), lambda g, _n=nb: (g + _n, 0)),
         _rows(_BE, 128), _rows(_BE, 14), _rows(_BE, 1),
         _full((14, 128)), _full((1, 128)), _full((1, 16)), _full((16, 128)),
         _full((1, 128)), _full((128, 128)), _full((1, 128)),
         _full((128, 128)), _full((1, 128)), _full((128, 128)),
         _full((1, 128)), _full((128, 128)), _full((1, 128))],
        [_rows(_BE, 128), _rows(_BE, 128), _rows(_BE, 128)],
    )


_Z = np.pi * (np.arange(1, NUM_RADIAL + 1)[None, :]
              + 0.5 * np.arange(NUM_SPHERICAL)[:, None])


def _tri_sel():
    sel = np.zeros((80, 288), np.float32)
    for l in range(6):
        for m in range(16):
            c = 16 * l + m
            k = 2 * m + l + 2
            sel[k - 2, c] = 1.0
            sel[36 + k - 2, 96 + c] = 1.0
            sel[77 if l == 0 else 71 + l, 192 + c] = 1.0
    return sel


def _tri_stage1(g1, gd_l, ang_l, lp):
    nr = _BT // 128

    def body(g_r, gd_r, a_r, z_r, sel_r, s1_w, s2_w, up_w, up_b, u_r, sbf_r):
        dtl = gd_r[0] * (1.0 / CUTOFF)
        th = dtl * (np.pi / 2.0)
        s1 = jnp.sin(th)
        c1 = jnp.cos(th)
        t = jnp.cos(a_r[0])
        ps = [jnp.ones_like(t), t]
        for l in range(1, 6):
            ps.append(((2.0 * l + 1.0) * t * ps[l] - l * ps[l - 1]) / (l + 1.0))
        two_c1 = 2.0 * c1
        sk = {1: s1}
        ck = {1: c1}
        sk[2] = two_c1 * s1
        ck[2] = two_c1 * c1 - 1.0
        for k in range(3, 38):
            sk[k] = two_c1 * sk[k - 1] - sk[k - 2]
            ck[k] = two_c1 * ck[k - 1] - ck[k - 2]
        arrs = ([sk[k] for k in range(2, 38)] + [ck[k] for k in range(2, 38)]
                + ps[1:6] + [jnp.ones_like(t), dtl, jnp.zeros_like(t)])
        m3 = jnp.stack(arrs, axis=0)
        t_all = jnp.concatenate(
            [m3[:, rr, :].T for rr in range(nr)], axis=0)
        big = _lin(t_all, sel_r[...])
        sinx = big[:, 0:96]
        cosx = big[:, 96:192]
        pmat = big[:, 192:288]
        dt = t_all[:, 78:79]
        x = dt * z_r[...]
        inv = 1.0 / x
        j0 = sinx * inv
        j1 = (sinx * inv - cosx) * inv
        parts = [j0[:, 0:16], j1[:, 16:32]]
        jm1, jc = j0, j1
        for ll in range(1, NUM_SPHERICAL - 1):
            jn = (2.0 * ll + 1.0) * inv * jc - jm1
            jm1, jc = jc, jn
            parts.append(jc[:, 16 * (ll + 1):16 * (ll + 2)])
        sbf = jnp.concatenate(parts, axis=1) * pmat
        sbf_r[...] = sbf.astype(jnp.bfloat16)
        sh = _act(_lin(_act(_lin(sbf, s1_w[...])), s2_w[...]))
        u_r[...] = _act(
            _lin(g_r[...].astype(jnp.float32) * sh, up_w[...], up_b[...])
        ).astype(jnp.bfloat16)

    return _tc_call(
        body,
        (N_TRIPLETS // _BT,),
        (
            jax.ShapeDtypeStruct((N_TRIPLETS, HIDDEN), jnp.bfloat16),
            jax.ShapeDtypeStruct((N_TRIPLETS, 96), jnp.bfloat16),
        ),
        (g1, gd_l, ang_l, jnp.asarray(_Z.reshape(1, -1), jnp.float32),
         jnp.asarray(_tri_sel()),
         lp["lin_sbf1_w"], lp["lin_sbf2_w"], lp["up_w"],
         lp["up_b"].reshape(1, HIDDEN)),
        [_rows(_BT, 128),
         pl.BlockSpec((1, _BT // 128, 128), lambda g: (g, 0, 0)),
         pl.BlockSpec((1, _BT // 128, 128), lambda g: (g, 0, 0)),
         _full((1, 96)), _full((80, 288)),
         _full((96, 128)), _full((128, 128)),
         _full((128, 128)), _full((1, 128))],
        [_rows(_BT, 128), _rows(_BT, 96)],
    )


def _tri_stage2(g2, sbf, lp):
    def body(g_r, sbf_r, s1_w, s2_w, up_w, up_b, u_r):
        sh = _act(_lin(_act(_lin(sbf_r[...], s1_w[...])), s2_w[...]))
        u_r[...] = _act(
            _lin(g_r[...].astype(jnp.float32) * sh, up_w[...], up_b[...])
        ).astype(jnp.bfloat16)

    return _tc_call(
        body,
        (N_TRIPLETS // _BT,),
        jax.ShapeDtypeStruct((N_TRIPLETS, HIDDEN), jnp.bfloat16),
        (g2, sbf, lp["lin_sbf1_w"], lp["lin_sbf2_w"], lp["up_w"],
         lp["up_b"].reshape(1, HIDDEN)),
        [_rows(_BT, 128), _rows(_BT, 96), _full((96, 128)), _full((128, 128)),
         _full((128, 128)), _full((1, 128))],
        _rows(_BT, 128),
    )


def _edge_stage2(agg1, msg0, rbfe, lp1, lp2):
    def body(ag_r, m0_r, re_r, r1_w, r1_b, r2_w, r2_b, kj_w, kj_b, rb_w,
             rb_b, dn_w, dn_b, m1_r, xk_r):
        ag = ag_r[...].astype(jnp.float32)
        m = ag + _act(_lin(_act(_lin(ag, r1_w[...], r1_b[...])), r2_w[...], r2_b[...]))
        m1 = m + m0_r[...]
        m1_r[...] = m1
        xkj = _act(_lin(m1, kj_w[...], kj_b[...]))
        xkj = xkj * _act(_lin(re_r[...], rb_w[...], rb_b[...]))
        xk_r[...] = _act(_lin(xkj, dn_w[...], dn_b[...]))

    return _tc_call(
        body,
        (N_EDGES // _BE,),
        (
            jax.ShapeDtypeStruct((N_EDGES, HIDDEN), jnp.float32),
            jax.ShapeDtypeStruct((N_EDGES, HIDDEN), jnp.float32),
        ),
        (agg1, msg0, rbfe,
         lp1["res1_w"], lp1["res1_b"].reshape(1, HIDDEN),
         lp1["res2_w"], lp1["res2_b"].reshape(1, HIDDEN),
         lp2["lin_kj_w"], lp2["lin_kj_b"].reshape(1, HIDDEN),
         lp2["lin_rbf2_w"], lp2["lin_rbf2_b"].reshape(1, HIDDEN),
         lp2["down_w"], lp2["down_b"].reshape(1, HIDDEN)),
        [_rows(_BE, 128)] * 3
        + [_full((128, 128)), _full((1, 128))] * 5,
        [_rows(_BE, 128), _rows(_BE, 128)],
    )


def _edge_stage3(agg2, msg1, lp2):
    def body(ag_r, m1_r, r1_w, r1_b, r2_w, r2_b, m2_r):
        ag = ag_r[...].astype(jnp.float32)
        m = ag + _act(_lin(_act(_lin(ag, r1_w[...], r1_b[...])), r2_w[...], r2_b[...]))
        m2_r[...] = m + m1_r[...]

    return _tc_call(
        body,
        (N_EDGES // _BE,),
        jax.ShapeDtypeStruct((N_EDGES, HIDDEN), jnp.float32),
        (agg2, msg1,
         lp2["res1_w"], lp2["res1_b"].reshape(1, HIDDEN),
         lp2["res2_w"], lp2["res2_b"].reshape(1, HIDDEN)),
        [_rows(_BE, 128)] * 2 + [_full((128, 128)), _full((1, 128))] * 2,
        _rows(_BE, 128),
    )


def _node_out(af, parts, p):
    wo_a = p["W_o_w"][:133]
    wo_m = p["W_o_w"][133:]
    bo = p["W_o_b"].reshape(1, HIDDEN)

    def body(af_r, pt_r, wa_r, wm_r, bo_r, o_r):
        am = pt_r[0] + pt_r[1]
        o_r[...] = _act(_lin(af_r[...], wa_r[...]) + _lin(am, wm_r[...]) + bo_r[...])

    return _tc_call(
        body,
        (N_NODES // _BN,),
        jax.ShapeDtypeStruct((N_NODES, HIDDEN), jnp.float32),
        (af, parts, wo_a, wo_m, bo),
        [_rows(_BN, 133),
         pl.BlockSpec((2, _BN, 128), lambda g: (0, g, 0)),
         _full((133, 128)), _full((128, 128)), _full((1, 128))],
        _rows(_BN, 128),
    )


def kernel(atom_feature, edge_feature, dist, angle, i, j, idx_kj, idx_ji,
           incomebond_edge_ids, incomebond_index_to_atom, params):
    p = params
    lp1, lp2 = p["layers"][0], p["layers"][1]
    dist2 = dist.reshape(N_EDGES, 1)
    ang_l = angle.reshape(-1, _BT // 128, 128)

    t_a, t_b2, t_i = _node_tables(atom_feature, p)
    tab_j = jnp.concatenate([t_a, t_b2], axis=0)
    jj = jnp.concatenate([j, j + N_NODES])
    gab = _sc_gather(tab_j, jj, 128, 2 * N_EDGES)
    g_ti = _sc_gather(t_i, i, 128, N_EDGES)
    msg0, rbfe, xk1 = _edge_stage1(gab, g_ti, edge_feature, dist2, p, lp1)
    gd_l = _sc_dist_gather(dist, idx_kj).reshape(-1, _BT // 128, 128)
    g1 = _sc_gather(xk1, idx_kj, 128, N_TRIPLETS)
    u1, sbf = _tri_stage1(g1, gd_l, ang_l, lp1)
    agg1 = _sc_scatter_add(u1, idx_kj)
    msg1, xk2 = _edge_stage2(agg1, msg0, rbfe, lp1, lp2)
    g2 = _sc_gather(xk2, idx_kj, 128, N_TRIPLETS)
    u2 = _tri_stage2(g2, sbf, lp2)
    agg2 = _sc_scatter_add(u2, idx_kj)
    msg2 = _edge_stage3(agg2, msg1, lp2)
    parts = _sc_atom_agg(msg2, incomebond_edge_ids, incomebond_index_to_atom)
    return _node_out(atom_feature, parts, p)
